# baseline (speedup 1.0000x reference)
import jax
import jax.numpy as jnp
from jax.experimental import pallas as pl
from jax.experimental.pallas import tpu as pltpu

MARGIN = 1.0
EPS = 1e-6
_BLK = 256


def _loss_kernel(emb_ref, tgt_ref, out_ref):
    e = emb_ref[...]
    t = tgt_ref[...].reshape(-1, 1)
    n, d = e.shape

    sq = jnp.sum(e * e, axis=1, keepdims=True)
    s = jnp.sum(e, axis=1, keepdims=True)

    a = sq + (0.5 * d * EPS * EPS)
    p = (2.0 * EPS) * s
    u = a + p
    v = a - p
    ones = jnp.ones((n, 1), jnp.float32)
    lhs = jnp.concatenate([e * -2.0, u, ones], axis=1)
    rhs = jnp.concatenate([e, ones, v], axis=1)

    t_row = t.reshape(1, -1)
    acc = jnp.zeros((_BLK, _BLK), jnp.float32)
    for bi in range(0, n, _BLK):
        for bj in range(bi, n, _BLK):
            d2 = jax.lax.dot_general(
                lhs[bi:bi + _BLK, :], rhs[bj:bj + _BLK, :],
                (((1,), (1,)), ((), ())),
                preferred_element_type=jnp.float32,
            )
            d2 = jnp.maximum(d2, 1e-12)
            dist = d2 * jax.lax.rsqrt(d2)
            neg = jnp.maximum(MARGIN - dist, 0.0)
            pos_mask = t[bi:bi + _BLK, :] == t_row[:, bj:bj + _BLK]
            loss = jnp.where(pos_mask, d2, neg * neg)
            if bi == bj:
                row = jax.lax.broadcasted_iota(jnp.int32, (_BLK, _BLK), 0)
                col = jax.lax.broadcasted_iota(jnp.int32, (_BLK, _BLK), 1)
                loss = jnp.where(col > row, loss, 0.0)
            acc = acc + loss

    out_ref[0] = jnp.sum(acc)


def kernel(embeddings, target):
    n = target.shape[0]
    loss_sum = pl.pallas_call(
        _loss_kernel,
        out_shape=jax.ShapeDtypeStruct((1,), jnp.float32),
        out_specs=pl.BlockSpec(memory_space=pltpu.SMEM),
    )(embeddings, target)
    n_pairs = jnp.asarray(n * (n - 1) // 2, dtype=jnp.int32)
    return (loss_sum[0], n_pairs)


# speedup vs baseline: 1.0219x; 1.0219x over previous
import jax
import jax.numpy as jnp
from jax.experimental import pallas as pl

MARGIN = 1.0
EPS = 1e-6
_BLK = 256


def _loss_kernel(emb_ref, tgt_ref, out_ref):
    e = emb_ref[...]
    t = tgt_ref[...].reshape(-1, 1)
    n, d = e.shape

    sq = jnp.sum(e * e, axis=1, keepdims=True)
    s = jnp.sum(e, axis=1, keepdims=True)

    a = sq + (0.5 * d * EPS * EPS)
    p = (2.0 * EPS) * s
    u = a + p
    v = a - p
    ones = jnp.ones((n, 1), jnp.float32)
    lhs = jnp.concatenate([e * -2.0, u, ones], axis=1)
    rhs = jnp.concatenate([e, ones, v], axis=1)

    t_row = t.reshape(1, -1)
    acc = jnp.zeros((_BLK, _BLK), jnp.float32)
    for bi in range(0, n, _BLK):
        for bj in range(bi, n, _BLK):
            d2 = jax.lax.dot_general(
                lhs[bi:bi + _BLK, :], rhs[bj:bj + _BLK, :],
                (((1,), (1,)), ((), ())),
                preferred_element_type=jnp.float32,
            )
            d2 = jnp.maximum(d2, 1e-12)
            dist = d2 * jax.lax.rsqrt(d2)
            neg = jnp.maximum(MARGIN - dist, 0.0)
            pos_mask = t[bi:bi + _BLK, :] == t_row[:, bj:bj + _BLK]
            loss = jnp.where(pos_mask, d2, neg * neg)
            if bi == bj:
                row = jax.lax.broadcasted_iota(jnp.int32, (_BLK, _BLK), 0)
                col = jax.lax.broadcasted_iota(jnp.int32, (_BLK, _BLK), 1)
                loss = jnp.where(col > row, loss, 0.0)
            acc = acc + loss

    out_ref[...] = jnp.sum(acc, keepdims=True)


def kernel(embeddings, target):
    n = target.shape[0]
    loss_sum = pl.pallas_call(
        _loss_kernel,
        out_shape=jax.ShapeDtypeStruct((1, 1), jnp.float32),
    )(embeddings, target)
    n_pairs = jnp.asarray(n * (n - 1) // 2, dtype=jnp.int32)
    return (loss_sum[0, 0], n_pairs)
